# Initial kernel scaffold; baseline (speedup 1.0000x reference)
import jax
import jax.numpy as jnp
from jax.experimental import pallas as pl


def kernel(data):
    raise NotImplementedError("write your pallas kernel here")


# speedup vs baseline: 36.9216x; 36.9216x over previous
import functools

import numpy as np
import jax
import jax.numpy as jnp
from jax.experimental import pallas as pl
from jax.experimental.pallas import tpu as pltpu

_ALPHA = 8.903838912968741
_P = 6
_RANK = 36.0
_ITERS = 12

_ann = np.ones((13, 13), dtype=bool)
_ann[4:9, 4:9] = False
_OFFS = [(int(i), int(j)) for i, j in zip(*np.nonzero(_ann))]


def _cfar_body(pad_ref, out_ref):
    V, R = out_ref.shape

    def step(_, carry):
        lo, half = carry
        t = lo + half
        cnt = jnp.zeros((V, R), jnp.float32)
        for di, dj in _OFFS:
            w = pad_ref[di:di + V, dj:dj + R]
            cnt = cnt + (w >= t).astype(jnp.float32)
        lo = jnp.where(cnt >= _RANK, t, lo)
        return lo, half * 0.5

    lo, half = jax.lax.fori_loop(
        0, _ITERS, step, (jnp.zeros((V, R), jnp.float32), jnp.float32(0.5)))
    out_ref[:, :] = (lo + half) * _ALPHA


def kernel(data):
    b, V, R = data.shape
    padded = jnp.pad(data[0], ((_P, _P), (_P, _P)), mode="wrap")
    out = pl.pallas_call(
        _cfar_body,
        out_shape=jax.ShapeDtypeStruct((V, R), jnp.float32),
        in_specs=[pl.BlockSpec(memory_space=pltpu.VMEM)],
        out_specs=pl.BlockSpec(memory_space=pltpu.VMEM),
    )(padded)
    return out
